# baseline (speedup 1.0000x reference)
import functools

import jax
import jax.numpy as jnp
from jax import lax
from jax.experimental import pallas as pl
from jax.experimental.pallas import tpu as pltpu
from jax.experimental.pallas import tpu_sc as plsc

N_USERS = 50000
N_ITEMS = 50000
N = N_USERS + N_ITEMS
E = 3200000
D = 16
N_LAYERS = 3
B = 4096

NC = 2
NS = 16
NW = NC * NS
L = 16

CHUNK = 128
KB = 4
GS = 2
CPW0 = 800
CPW1 = 800
NGRP0 = CPW0 // KB
NGRP1 = CPW1 // KB
TOT_GRP = NS * (NGRP0 + NGRP1)
EP = TOT_GRP * KB * CHUNK
ROWS_PER_SUB = N // NS

_MESH = plsc.VectorSubcoreMesh(
    core_axis_name="c", subcore_axis_name="s", num_cores=NC, num_subcores=NS
)


@functools.partial(
    pl.kernel,
    out_type=(
        jax.ShapeDtypeStruct((N, D), jnp.float32),
        jax.ShapeDtypeStruct((N, D), jnp.float32),
    ),
    mesh=_MESH,
    compiler_params=pltpu.CompilerParams(use_tc_tiling_on_sc=False, needs_layout_passes=False),
    scratch_types=[
        pltpu.VMEM((3, KB, GS, CHUNK // GS), jnp.int32),
        pltpu.VMEM((3, KB, CHUNK), jnp.int32),
        pltpu.VMEM((3, KB, CHUNK), jnp.float32),
        pltpu.VMEM((KB, CHUNK, D), jnp.float32),
        pltpu.VMEM((KB, CHUNK, D), jnp.float32),
        pltpu.VMEM((625, D), jnp.float32),
        pltpu.VMEM_SHARED((N, D), jnp.float32),
        pltpu.SemaphoreType.DMA((KB,)),
        pltpu.SemaphoreType.DMA((KB,)),
        pltpu.SemaphoreType.DMA,
    ],
)
def _spmm_layer(e_in, cols3, rows3, adj3, p0, p1,
                colb, rowb, adjb, gbi, gbo, zbuf, acc, gsem, ssem, isem):
    cid = lax.axis_index("c")
    sid = lax.axis_index("s")
    w = cid * NS + sid
    r0 = sid * ROWS_PER_SUB

    zv = jnp.zeros((L,), jnp.float32)

    def zrow(r, carry):
        zbuf[r, :] = zv
        return carry

    with jax.named_scope("ph_zero"):
        lax.fori_loop(0, 625, zrow, 0)
        for zb in range(ROWS_PER_SUB // 625):
            pltpu.sync_copy(zbuf, acc.at[pl.ds(r0 + zb * 625, 625)])
        plsc.subcore_barrier()

    ngrp = jnp.where(cid == 0, NGRP0, NGRP1)
    gbase = jnp.where(cid == 0, sid * NGRP0, NS * NGRP0 + sid * NGRP1)

    def scale_chunk(slot, j):
        for g in range(CHUNK // L):
            wv = adjb[slot, j, pl.ds(g * L, L)]
            for t in range(L):
                e = g * L + t
                gbo[j, e, :] = gbi[j, e, :] * wv[t]

    def issue_idx_loads(grp, slot):
        pltpu.async_copy(cols3.at[gbase + grp], colb.at[slot], isem)
        pltpu.async_copy(rows3.at[gbase + grp], rowb.at[slot], isem)
        pltpu.async_copy(adj3.at[gbase + grp], adjb.at[slot], isem)

    def wait_idx_loads():
        pltpu.make_async_copy(cols3.at[gbase], colb.at[0], isem).wait()
        pltpu.make_async_copy(rows3.at[gbase], rowb.at[0], isem).wait()
        pltpu.make_async_copy(adj3.at[gbase], adjb.at[0], isem).wait()

    def issue_gather(grp_slot, j):
        h = CHUNK // GS
        for gsplit in range(GS):
            pltpu.async_copy(
                e_in.at[colb.at[grp_slot, j, gsplit]],
                gbi.at[j, pl.ds(gsplit * h, h)], gsem.at[j])

    def wait_gather(j):
        pltpu.make_async_copy(e_in.at[colb.at[0, j, 0]], gbi.at[j],
                              gsem.at[j]).wait()

    def issue_scatter(grp_slot, j):
        pltpu.async_copy(gbo.at[j], acc.at[rowb.at[grp_slot, j]],
                         ssem.at[j], add=True)

    def wait_scatter(j):
        pltpu.make_async_copy(gbo.at[j], acc.at[rowb.at[0, j]],
                              ssem.at[j]).wait()

    pltpu.sync_copy(cols3.at[gbase], colb.at[0])
    pltpu.sync_copy(rows3.at[gbase], rowb.at[0])
    pltpu.sync_copy(adj3.at[gbase], adjb.at[0])
    for j in range(KB):
        issue_gather(0, j)
    issue_idx_loads(1, 1)

    def grp_body(g, carry):
        cur = lax.rem(g, 3)
        nxt = lax.rem(g + 1, 3)
        over = lax.rem(g + 2, 3)
        wait_idx_loads()
        for j in range(KB):
            wait_gather(j)

            @pl.when(g > 0)
            def _():
                wait_scatter(j)

            scale_chunk(cur, j)
            issue_scatter(cur, j)
            issue_gather(nxt, j)
        issue_idx_loads(jnp.minimum(g + 2, ngrp - 1), over)
        return carry

    with jax.named_scope("ph_main"):
        lax.fori_loop(0, ngrp, grp_body, 0)
    with jax.named_scope("ph_dr_idx"):
        wait_idx_loads()
    with jax.named_scope("ph_dr_g"):
        for j in range(KB):
            wait_gather(j)
    with jax.named_scope("ph_dr_s"):
        for j in range(KB):
            wait_scatter(j)
    with jax.named_scope("ph_bar"):
        plsc.subcore_barrier()

    with jax.named_scope("ph_wb"):
        @pl.when(cid == 0)
        def _():
            pltpu.sync_copy(acc.at[pl.ds(r0, ROWS_PER_SUB)],
                            p0.at[pl.ds(r0, ROWS_PER_SUB)])

        @pl.when(cid == 1)
        def _():
            pltpu.sync_copy(acc.at[pl.ds(r0, ROWS_PER_SUB)],
                            p1.at[pl.ds(r0, ROWS_PER_SUB)])


RPW = N // NW
RB = 625
NBLK = RPW // RB


@functools.partial(
    pl.kernel,
    out_type=(
        jax.ShapeDtypeStruct((N, D), jnp.float32),
        jax.ShapeDtypeStruct((N, D), jnp.float32),
    ),
    mesh=_MESH,
    compiler_params=pltpu.CompilerParams(use_tc_tiling_on_sc=False,
                                         needs_layout_passes=False),
    scratch_types=[
        pltpu.VMEM((RB, D), jnp.float32),
        pltpu.VMEM((RB, D), jnp.float32),
        pltpu.VMEM((RB, D), jnp.float32),
    ],
)
def _combine(p0, p1, sum_in, e_next, sum_out, b0, b1, bs):
    cid = lax.axis_index("c")
    sid = lax.axis_index("s")
    w = cid * NS + sid
    r0 = w * RPW
    for blk in range(NBLK):
        base = r0 + blk * RB
        pltpu.sync_copy(p0.at[pl.ds(base, RB)], b0)
        pltpu.sync_copy(p1.at[pl.ds(base, RB)], b1)
        pltpu.sync_copy(sum_in.at[pl.ds(base, RB)], bs)

        def row_body(r, carry):
            e = b0[r, :] + b1[r, :]
            b0[r, :] = e
            bs[r, :] = bs[r, :] + e
            return carry

        lax.fori_loop(0, RB, row_body, 0)
        pltpu.sync_copy(b0, e_next.at[pl.ds(base, RB)])
        pltpu.sync_copy(bs, sum_out.at[pl.ds(base, RB)])

PPW = B // NW


@functools.partial(
    pl.kernel,
    out_type=jax.ShapeDtypeStruct((B,), jnp.float32),
    mesh=_MESH,
    compiler_params=pltpu.CompilerParams(use_tc_tiling_on_sc=False, needs_layout_passes=False),
    scratch_types=[
        pltpu.VMEM((PPW,), jnp.int32),
        pltpu.VMEM((PPW,), jnp.int32),
        pltpu.VMEM((PPW, D), jnp.float32),
        pltpu.VMEM((PPW, D), jnp.float32),
        pltpu.VMEM((PPW,), jnp.float32),
    ],
)
def _final_logits(sum_in, uidx2, iidx2, out, uib, iib, ub, ib, lb):
    cid = lax.axis_index("c")
    sid = lax.axis_index("s")
    w = cid * NS + sid
    pltpu.sync_copy(uidx2.at[w], uib)
    pltpu.sync_copy(iidx2.at[w], iib)
    pltpu.sync_copy(sum_in.at[uib], ub)
    pltpu.sync_copy(sum_in.at[iib], ib)
    lanes = lax.iota(jnp.int32, L)
    for g in range(PPW // L):
        pids = jnp.full((L,), g * L, jnp.int32) + lanes
        accv = jnp.zeros((L,), jnp.float32)
        for dd in range(D):
            dv = jnp.full((L,), dd, jnp.int32)
            uc = plsc.load_gather(ub, [pids, dv])
            ic = plsc.load_gather(ib, [pids, dv])
            accv = accv + uc * ic
        lb[pl.ds(g * L, L)] = accv * jnp.float32(1.0 / 16.0)
    off = pl.multiple_of(w * PPW, 8)
    pltpu.sync_copy(lb, out.at[pl.ds(off, PPW)])


def kernel(user_idx, item_idx, adj_indices, adj_data, user_table, item_table):
    e0 = jnp.concatenate([user_table, item_table], axis=0)
    rows = adj_indices[:, 0]
    cols = adj_indices[:, 1]
    pad = EP - E
    rows2 = jnp.pad(rows, (0, pad)).reshape(TOT_GRP, KB, CHUNK)
    cols2 = jnp.pad(cols, (0, pad)).reshape(TOT_GRP, KB, GS, CHUNK // GS)
    adj2 = jnp.pad(adj_data, (0, pad)).reshape(TOT_GRP, KB, CHUNK)

    e_k = e0
    esum = e0
    for _ in range(N_LAYERS):
        p0, p1 = _spmm_layer(e_k, cols2, rows2, adj2)
        e_k, esum = _combine(p0, p1, esum)

    uidx2 = user_idx.reshape(NW, PPW)
    iidx2 = (item_idx + N_USERS).reshape(NW, PPW)
    return _final_logits(esum, uidx2, iidx2)


# speedup vs baseline: 1.9790x; 1.9790x over previous
import functools

import jax
import jax.numpy as jnp
from jax import lax
from jax.experimental import pallas as pl
from jax.experimental.pallas import tpu as pltpu
from jax.experimental.pallas import tpu_sc as plsc

N_USERS = 50000
N_ITEMS = 50000
N = N_USERS + N_ITEMS
E = 3200000
D = 16
N_LAYERS = 3
B = 4096

NC = 2
NS = 16
NW = NC * NS
L = 16

CHUNK = 128
KB = 4
GS = 1
CPW0 = 800
CPW1 = 800
NGRP0 = CPW0 // KB
NGRP1 = CPW1 // KB
TOT_GRP = NS * (NGRP0 + NGRP1)
EP = TOT_GRP * KB * CHUNK
ROWS_PER_SUB = N // NS

_MESH = plsc.VectorSubcoreMesh(
    core_axis_name="c", subcore_axis_name="s", num_cores=NC, num_subcores=NS
)


@functools.partial(
    pl.kernel,
    out_type=(
        jax.ShapeDtypeStruct((N, D), jnp.float32),
        jax.ShapeDtypeStruct((N, D), jnp.float32),
    ),
    mesh=_MESH,
    compiler_params=pltpu.CompilerParams(use_tc_tiling_on_sc=False, needs_layout_passes=False),
    scratch_types=[
        pltpu.VMEM((3, KB, GS, CHUNK // GS), jnp.int32),
        pltpu.VMEM((3, KB, CHUNK), jnp.int32),
        pltpu.VMEM((3, KB, CHUNK), jnp.float32),
        pltpu.VMEM((KB, CHUNK, D), jnp.float32),
        pltpu.VMEM((KB, CHUNK, D), jnp.float32),
        pltpu.VMEM((625, D), jnp.float32),
        pltpu.VMEM_SHARED((N, D), jnp.float32),
        pltpu.SemaphoreType.DMA((KB,)),
        pltpu.SemaphoreType.DMA((KB,)),
        pltpu.SemaphoreType.DMA,
    ],
)
def _spmm_layer(e_in, cols3, rows3, adj3, p0, p1,
                colb, rowb, adjb, gbi, gbo, zbuf, acc, gsem, ssem, isem):
    cid = lax.axis_index("c")
    sid = lax.axis_index("s")
    w = cid * NS + sid
    r0 = sid * ROWS_PER_SUB

    zv = jnp.zeros((L,), jnp.float32)

    def zrow(r, carry):
        zbuf[r, :] = zv
        return carry

    with jax.named_scope("ph_zero"):
        lax.fori_loop(0, 625, zrow, 0)
        for zb in range(ROWS_PER_SUB // 625):
            pltpu.sync_copy(zbuf, acc.at[pl.ds(r0 + zb * 625, 625)])
        plsc.subcore_barrier()

    ngrp = jnp.where(cid == 0, NGRP0, NGRP1)
    gbase = jnp.where(cid == 0, sid * NGRP0, NS * NGRP0 + sid * NGRP1)

    def scale_chunk(slot, j):
        for g in range(CHUNK // L):
            wv = adjb[slot, j, pl.ds(g * L, L)]
            for t in range(L):
                e = g * L + t
                gbo[j, e, :] = gbi[j, e, :] * wv[t]

    def issue_idx_loads(grp, slot):
        pltpu.async_copy(cols3.at[gbase + grp], colb.at[slot], isem)
        pltpu.async_copy(rows3.at[gbase + grp], rowb.at[slot], isem)
        pltpu.async_copy(adj3.at[gbase + grp], adjb.at[slot], isem)

    def wait_idx_loads():
        pltpu.make_async_copy(cols3.at[gbase], colb.at[0], isem).wait()
        pltpu.make_async_copy(rows3.at[gbase], rowb.at[0], isem).wait()
        pltpu.make_async_copy(adj3.at[gbase], adjb.at[0], isem).wait()

    def issue_gather(grp_slot, j):
        h = CHUNK // GS
        for gsplit in range(GS):
            pltpu.async_copy(
                e_in.at[colb.at[grp_slot, j, gsplit]],
                gbi.at[j, pl.ds(gsplit * h, h)], gsem.at[j])

    def wait_gather(j):
        pltpu.make_async_copy(e_in.at[colb.at[0, j, 0]], gbi.at[j],
                              gsem.at[j]).wait()

    def issue_scatter(grp_slot, j):
        pltpu.async_copy(gbo.at[j], acc.at[rowb.at[grp_slot, j]],
                         ssem.at[j], add=True)

    def wait_scatter(j):
        pltpu.make_async_copy(gbo.at[j], acc.at[rowb.at[0, j]],
                              ssem.at[j]).wait()

    pltpu.sync_copy(cols3.at[gbase], colb.at[0])
    pltpu.sync_copy(rows3.at[gbase], rowb.at[0])
    pltpu.sync_copy(adj3.at[gbase], adjb.at[0])
    for j in range(KB):
        issue_gather(0, j)
    issue_idx_loads(1, 1)

    def grp_body(g, carry):
        cur = lax.rem(g, 3)
        nxt = lax.rem(g + 1, 3)
        over = lax.rem(g + 2, 3)
        wait_idx_loads()
        for j in range(KB):
            wait_gather(j)

            @pl.when(g > 0)
            def _():
                wait_scatter(j)

            scale_chunk(cur, j)
            issue_scatter(cur, j)
            issue_gather(nxt, j)
        issue_idx_loads(jnp.minimum(g + 2, ngrp - 1), over)
        return carry

    with jax.named_scope("ph_main"):
        lax.fori_loop(0, ngrp, grp_body, 0)
    with jax.named_scope("ph_dr_idx"):
        wait_idx_loads()
    with jax.named_scope("ph_dr_g"):
        for j in range(KB):
            wait_gather(j)
    with jax.named_scope("ph_dr_s"):
        for j in range(KB):
            wait_scatter(j)
    with jax.named_scope("ph_bar"):
        plsc.subcore_barrier()

    with jax.named_scope("ph_wb"):
        @pl.when(cid == 0)
        def _():
            pltpu.sync_copy(acc.at[pl.ds(r0, ROWS_PER_SUB)],
                            p0.at[pl.ds(r0, ROWS_PER_SUB)])

        @pl.when(cid == 1)
        def _():
            pltpu.sync_copy(acc.at[pl.ds(r0, ROWS_PER_SUB)],
                            p1.at[pl.ds(r0, ROWS_PER_SUB)])


RPW = N // NW
RB = 625
NBLK = RPW // RB


@functools.partial(
    pl.kernel,
    out_type=(
        jax.ShapeDtypeStruct((N, D), jnp.float32),
        jax.ShapeDtypeStruct((N, D), jnp.float32),
    ),
    mesh=_MESH,
    compiler_params=pltpu.CompilerParams(use_tc_tiling_on_sc=False,
                                         needs_layout_passes=False),
    scratch_types=[
        pltpu.VMEM((RB, D), jnp.float32),
        pltpu.VMEM((RB, D), jnp.float32),
        pltpu.VMEM((RB, D), jnp.float32),
    ],
)
def _combine(p0, p1, sum_in, e_next, sum_out, b0, b1, bs):
    cid = lax.axis_index("c")
    sid = lax.axis_index("s")
    w = cid * NS + sid
    r0 = w * RPW
    for blk in range(NBLK):
        base = r0 + blk * RB
        pltpu.sync_copy(p0.at[pl.ds(base, RB)], b0)
        pltpu.sync_copy(p1.at[pl.ds(base, RB)], b1)
        pltpu.sync_copy(sum_in.at[pl.ds(base, RB)], bs)

        def row_body(r, carry):
            e = b0[r, :] + b1[r, :]
            b0[r, :] = e
            bs[r, :] = bs[r, :] + e
            return carry

        lax.fori_loop(0, RB, row_body, 0)
        pltpu.sync_copy(b0, e_next.at[pl.ds(base, RB)])
        pltpu.sync_copy(bs, sum_out.at[pl.ds(base, RB)])

PPW = B // NW


@functools.partial(
    pl.kernel,
    out_type=jax.ShapeDtypeStruct((B,), jnp.float32),
    mesh=_MESH,
    compiler_params=pltpu.CompilerParams(use_tc_tiling_on_sc=False, needs_layout_passes=False),
    scratch_types=[
        pltpu.VMEM((PPW,), jnp.int32),
        pltpu.VMEM((PPW,), jnp.int32),
        pltpu.VMEM((PPW, D), jnp.float32),
        pltpu.VMEM((PPW, D), jnp.float32),
        pltpu.VMEM((PPW,), jnp.float32),
    ],
)
def _final_logits(sum_in, uidx2, iidx2, out, uib, iib, ub, ib, lb):
    cid = lax.axis_index("c")
    sid = lax.axis_index("s")
    w = cid * NS + sid
    pltpu.sync_copy(uidx2.at[w], uib)
    pltpu.sync_copy(iidx2.at[w], iib)
    pltpu.sync_copy(sum_in.at[uib], ub)
    pltpu.sync_copy(sum_in.at[iib], ib)
    lanes = lax.iota(jnp.int32, L)
    for g in range(PPW // L):
        pids = jnp.full((L,), g * L, jnp.int32) + lanes
        accv = jnp.zeros((L,), jnp.float32)
        for dd in range(D):
            dv = jnp.full((L,), dd, jnp.int32)
            uc = plsc.load_gather(ub, [pids, dv])
            ic = plsc.load_gather(ib, [pids, dv])
            accv = accv + uc * ic
        lb[pl.ds(g * L, L)] = accv * jnp.float32(1.0 / 16.0)
    off = pl.multiple_of(w * PPW, 8)
    pltpu.sync_copy(lb, out.at[pl.ds(off, PPW)])


def kernel(user_idx, item_idx, adj_indices, adj_data, user_table, item_table):
    e0 = jnp.concatenate([user_table, item_table], axis=0)
    rows = adj_indices[:, 0]
    cols = adj_indices[:, 1]
    pad = EP - E
    spread = jnp.arange(pad, dtype=jnp.int32) % N
    rows2 = jnp.concatenate([rows, spread]).reshape(TOT_GRP, KB, CHUNK)
    cols2 = jnp.concatenate([cols, spread]).reshape(TOT_GRP, KB, GS, CHUNK // GS)
    adj2 = jnp.pad(adj_data, (0, pad)).reshape(TOT_GRP, KB, CHUNK)

    e_k = e0
    esum = e0
    for _ in range(N_LAYERS):
        p0, p1 = _spmm_layer(e_k, cols2, rows2, adj2)
        e_k, esum = _combine(p0, p1, esum)

    uidx2 = user_idx.reshape(NW, PPW)
    iidx2 = (item_idx + N_USERS).reshape(NW, PPW)
    return _final_logits(esum, uidx2, iidx2)


# speedup vs baseline: 2.1056x; 1.0640x over previous
import functools

import jax
import jax.numpy as jnp
from jax import lax
from jax.experimental import pallas as pl
from jax.experimental.pallas import tpu as pltpu
from jax.experimental.pallas import tpu_sc as plsc

N_USERS = 50000
N_ITEMS = 50000
N = N_USERS + N_ITEMS
E = 3200000
D = 16
N_LAYERS = 3
B = 4096

NC = 2
NS = 16
NW = NC * NS
L = 16

CHUNK = 128
KB = 5
ZB = 250
GS = 1
CPW0 = 800
CPW1 = 800
NGRP0 = CPW0 // KB
NGRP1 = CPW1 // KB
TOT_GRP = NS * (NGRP0 + NGRP1)
EP = TOT_GRP * KB * CHUNK
ROWS_PER_SUB = N // NS

_MESH = plsc.VectorSubcoreMesh(
    core_axis_name="c", subcore_axis_name="s", num_cores=NC, num_subcores=NS
)


@functools.partial(
    pl.kernel,
    out_type=(
        jax.ShapeDtypeStruct((N, D), jnp.float32),
        jax.ShapeDtypeStruct((N, D), jnp.float32),
    ),
    mesh=_MESH,
    compiler_params=pltpu.CompilerParams(use_tc_tiling_on_sc=False, needs_layout_passes=False),
    scratch_types=[
        pltpu.VMEM((3, KB, GS, CHUNK // GS), jnp.int32),
        pltpu.VMEM((3, KB, CHUNK), jnp.int32),
        pltpu.VMEM((3, KB, CHUNK), jnp.float32),
        pltpu.VMEM((KB, CHUNK, D), jnp.float32),
        pltpu.VMEM((KB, CHUNK, D), jnp.float32),
        pltpu.VMEM((ZB, D), jnp.float32),
        pltpu.VMEM_SHARED((N, D), jnp.float32),
        pltpu.SemaphoreType.DMA((KB,)),
        pltpu.SemaphoreType.DMA((KB,)),
        pltpu.SemaphoreType.DMA,
    ],
)
def _spmm_layer(e_in, cols3, rows3, adj3, p0, p1,
                colb, rowb, adjb, gbi, gbo, zbuf, acc, gsem, ssem, isem):
    cid = lax.axis_index("c")
    sid = lax.axis_index("s")
    w = cid * NS + sid
    r0 = sid * ROWS_PER_SUB

    zv = jnp.zeros((L,), jnp.float32)

    def zrow(r, carry):
        zbuf[r, :] = zv
        return carry

    lax.fori_loop(0, ZB, zrow, 0)
    for zb in range(ROWS_PER_SUB // ZB):
        pltpu.sync_copy(zbuf, acc.at[pl.ds(r0 + zb * ZB, ZB)])
    plsc.subcore_barrier()

    ngrp = jnp.where(cid == 0, NGRP0, NGRP1)
    gbase = jnp.where(cid == 0, sid * NGRP0, NS * NGRP0 + sid * NGRP1)

    def scale_chunk(slot, j):
        for g in range(CHUNK // L):
            wv = adjb[slot, j, pl.ds(g * L, L)]
            for t in range(L):
                e = g * L + t
                gbo[j, e, :] = gbi[j, e, :] * wv[t]

    def issue_idx_loads(grp, slot):
        pltpu.async_copy(cols3.at[gbase + grp], colb.at[slot], isem)
        pltpu.async_copy(rows3.at[gbase + grp], rowb.at[slot], isem)
        pltpu.async_copy(adj3.at[gbase + grp], adjb.at[slot], isem)

    def wait_idx_loads():
        pltpu.make_async_copy(cols3.at[gbase], colb.at[0], isem).wait()
        pltpu.make_async_copy(rows3.at[gbase], rowb.at[0], isem).wait()
        pltpu.make_async_copy(adj3.at[gbase], adjb.at[0], isem).wait()

    def issue_gather(grp_slot, j):
        h = CHUNK // GS
        for gsplit in range(GS):
            pltpu.async_copy(
                e_in.at[colb.at[grp_slot, j, gsplit]],
                gbi.at[j, pl.ds(gsplit * h, h)], gsem.at[j])

    def wait_gather(j):
        pltpu.make_async_copy(e_in.at[colb.at[0, j, 0]], gbi.at[j],
                              gsem.at[j]).wait()

    def issue_scatter(grp_slot, j):
        pltpu.async_copy(gbo.at[j], acc.at[rowb.at[grp_slot, j]],
                         ssem.at[j], add=True)

    def wait_scatter(j):
        pltpu.make_async_copy(gbo.at[j], acc.at[rowb.at[0, j]],
                              ssem.at[j]).wait()

    pltpu.sync_copy(cols3.at[gbase], colb.at[0])
    pltpu.sync_copy(rows3.at[gbase], rowb.at[0])
    pltpu.sync_copy(adj3.at[gbase], adjb.at[0])
    for j in range(KB):
        issue_gather(0, j)
    issue_idx_loads(1, 1)

    def grp_body(g, carry):
        cur = lax.rem(g, 3)
        nxt = lax.rem(g + 1, 3)
        over = lax.rem(g + 2, 3)
        wait_idx_loads()
        for j in range(KB):
            wait_gather(j)

            @pl.when(g > 0)
            def _():
                wait_scatter(j)

            scale_chunk(cur, j)
            issue_scatter(cur, j)
            issue_gather(nxt, j)
        issue_idx_loads(jnp.minimum(g + 2, ngrp - 1), over)
        return carry

    lax.fori_loop(0, ngrp, grp_body, 0)
    wait_idx_loads()
    for j in range(KB):
        wait_gather(j)
    for j in range(KB):
        wait_scatter(j)
    plsc.subcore_barrier()

    @pl.when(cid == 0)
    def _():
        pltpu.sync_copy(acc.at[pl.ds(r0, ROWS_PER_SUB)],
                        p0.at[pl.ds(r0, ROWS_PER_SUB)])

    @pl.when(cid == 1)
    def _():
        pltpu.sync_copy(acc.at[pl.ds(r0, ROWS_PER_SUB)],
                        p1.at[pl.ds(r0, ROWS_PER_SUB)])


RPW = N // NW
RB = 625
NBLK = RPW // RB


@functools.partial(
    pl.kernel,
    out_type=(
        jax.ShapeDtypeStruct((N, D), jnp.float32),
        jax.ShapeDtypeStruct((N, D), jnp.float32),
    ),
    mesh=_MESH,
    compiler_params=pltpu.CompilerParams(use_tc_tiling_on_sc=False,
                                         needs_layout_passes=False),
    scratch_types=[
        pltpu.VMEM((RB, D), jnp.float32),
        pltpu.VMEM((RB, D), jnp.float32),
        pltpu.VMEM((RB, D), jnp.float32),
    ],
)
def _combine(p0, p1, sum_in, e_next, sum_out, b0, b1, bs):
    cid = lax.axis_index("c")
    sid = lax.axis_index("s")
    w = cid * NS + sid
    r0 = w * RPW
    for blk in range(NBLK):
        base = r0 + blk * RB
        pltpu.sync_copy(p0.at[pl.ds(base, RB)], b0)
        pltpu.sync_copy(p1.at[pl.ds(base, RB)], b1)
        pltpu.sync_copy(sum_in.at[pl.ds(base, RB)], bs)

        def row_body(r, carry):
            e = b0[r, :] + b1[r, :]
            b0[r, :] = e
            bs[r, :] = bs[r, :] + e
            return carry

        lax.fori_loop(0, RB, row_body, 0)
        pltpu.sync_copy(b0, e_next.at[pl.ds(base, RB)])
        pltpu.sync_copy(bs, sum_out.at[pl.ds(base, RB)])

PPW = B // NW


@functools.partial(
    pl.kernel,
    out_type=jax.ShapeDtypeStruct((B,), jnp.float32),
    mesh=_MESH,
    compiler_params=pltpu.CompilerParams(use_tc_tiling_on_sc=False, needs_layout_passes=False),
    scratch_types=[
        pltpu.VMEM((PPW,), jnp.int32),
        pltpu.VMEM((PPW,), jnp.int32),
        pltpu.VMEM((PPW, D), jnp.float32),
        pltpu.VMEM((PPW, D), jnp.float32),
        pltpu.VMEM((PPW,), jnp.float32),
    ],
)
def _final_logits(sum_in, uidx2, iidx2, out, uib, iib, ub, ib, lb):
    cid = lax.axis_index("c")
    sid = lax.axis_index("s")
    w = cid * NS + sid
    pltpu.sync_copy(uidx2.at[w], uib)
    pltpu.sync_copy(iidx2.at[w], iib)
    pltpu.sync_copy(sum_in.at[uib], ub)
    pltpu.sync_copy(sum_in.at[iib], ib)
    lanes = lax.iota(jnp.int32, L)
    for g in range(PPW // L):
        pids = jnp.full((L,), g * L, jnp.int32) + lanes
        accv = jnp.zeros((L,), jnp.float32)
        for dd in range(D):
            dv = jnp.full((L,), dd, jnp.int32)
            uc = plsc.load_gather(ub, [pids, dv])
            ic = plsc.load_gather(ib, [pids, dv])
            accv = accv + uc * ic
        lb[pl.ds(g * L, L)] = accv * jnp.float32(1.0 / 16.0)
    off = pl.multiple_of(w * PPW, 8)
    pltpu.sync_copy(lb, out.at[pl.ds(off, PPW)])


def kernel(user_idx, item_idx, adj_indices, adj_data, user_table, item_table):
    e0 = jnp.concatenate([user_table, item_table], axis=0)
    rows = adj_indices[:, 0]
    cols = adj_indices[:, 1]
    pad = EP - E
    spread = jnp.arange(pad, dtype=jnp.int32) % N
    rows2 = jnp.concatenate([rows, spread]).reshape(TOT_GRP, KB, CHUNK)
    cols2 = jnp.concatenate([cols, spread]).reshape(TOT_GRP, KB, GS, CHUNK // GS)
    adj2 = jnp.pad(adj_data, (0, pad)).reshape(TOT_GRP, KB, CHUNK)

    e_k = e0
    esum = e0
    for _ in range(N_LAYERS):
        p0, p1 = _spmm_layer(e_k, cols2, rows2, adj2)
        e_k, esum = _combine(p0, p1, esum)

    uidx2 = user_idx.reshape(NW, PPW)
    iidx2 = (item_idx + N_USERS).reshape(NW, PPW)
    return _final_logits(esum, uidx2, iidx2)
